# baseline (speedup 1.0000x reference)
import jax
import jax.numpy as jnp
from jax.experimental import pallas as pl
from jax.experimental.pallas import tpu as pltpu

_TS = 26
_TOT = 126
_REP = 20
_F = _TS + (_TOT - _TS) * _REP
_D = 64
_FLAT_ROWS = _F * _D // 128
_S_ROWS = _TS * _D // 128
_T_ROWS = (_TOT - _TS) * _D // 128
_BB = 8


def _body(tbl_ref, x_ref, o_ref, bias_ref):
    @pl.when(pl.program_id(0) == 0)
    def _init():
        tbl = tbl_ref[...]
        bias_ref[0:_TS] = tbl[0:_TS]
        ts = tbl[_TS:_TOT]
        for r in range(_REP):
            lo = _TS + r * (_TOT - _TS)
            bias_ref[lo:lo + (_TOT - _TS)] = ts

    o_ref[...] = x_ref[...] + bias_ref[...][None, :, :]


def kernel(X, table):
    B = X.shape[0]
    out = pl.pallas_call(
        _body,
        grid=(B // _BB,),
        in_specs=[
            pl.BlockSpec((_TOT, _D), lambda i: (0, 0)),
            pl.BlockSpec((_BB, _F, _D), lambda i: (i, 0, 0)),
        ],
        out_specs=pl.BlockSpec((_BB, _F, _D), lambda i: (i, 0, 0)),
        out_shape=jax.ShapeDtypeStruct(X.shape, X.dtype),
        scratch_shapes=[pltpu.VMEM((_F, _D), jnp.float32)],
    )(table, X)
    return out


# speedup vs baseline: 1.7720x; 1.7720x over previous
import jax
import jax.numpy as jnp
from jax.experimental import pallas as pl
from jax.experimental.pallas import tpu as pltpu

_TS = 26
_TOT = 126
_REP = 20
_F = _TS + (_TOT - _TS) * _REP
_D = 64
_FD = _F * _D
_S = _TS * _D
_T = (_TOT - _TS) * _D
_BB = 8


def _body(tbl_s_ref, tbl_t_ref, x_ref, o_ref, bias_ref):
    @pl.when(pl.program_id(0) == 0)
    def _init():
        bias_ref[:, 0:_S] = tbl_s_ref[...]
        ts = tbl_t_ref[...]
        for r in range(_REP):
            lo = _S + r * _T
            bias_ref[:, lo:lo + _T] = ts

    o_ref[...] = x_ref[...] + bias_ref[...]


def kernel(X, table):
    B = X.shape[0]
    x_flat = X.reshape(B, _FD)
    tbl_s = table[:_TS].reshape(1, _S)
    tbl_t = table[_TS:].reshape(1, _T)

    out = pl.pallas_call(
        _body,
        grid=(B // _BB,),
        in_specs=[
            pl.BlockSpec((1, _S), lambda i: (0, 0)),
            pl.BlockSpec((1, _T), lambda i: (0, 0)),
            pl.BlockSpec((_BB, _FD), lambda i: (i, 0)),
        ],
        out_specs=pl.BlockSpec((_BB, _FD), lambda i: (i, 0)),
        out_shape=jax.ShapeDtypeStruct((B, _FD), X.dtype),
        scratch_shapes=[pltpu.VMEM((1, _FD), jnp.float32)],
    )(tbl_s, tbl_t, x_flat)
    return out.reshape(X.shape)


# speedup vs baseline: 6.2081x; 3.5035x over previous
import functools

import jax
import jax.numpy as jnp
from jax.experimental import pallas as pl

_TS = 26
_TOT = 126
_REP = 20
_F = _TS + (_TOT - _TS) * _REP
_D = 64
_B = 1024
_FB = 32
_PAIRS = _FB // 2
_NPAIR = _F // 2
_LANE = 128


def _bias_index_map(j, i):
    p = jnp.minimum(i * _PAIRS + j, _NPAIR - 1)
    rh = jnp.where(p < _TS // 2, p, _TS // 2 + (p - _TS // 2) % 50)
    return rh, 0, 0


def _body(x_ref, *rest):
    b_refs = rest[:_PAIRS]
    o_ref = rest[_PAIRS]
    for j in range(_PAIRS):
        bias = jnp.concatenate([b_refs[j][...]] * (_B // _LANE), axis=2)
        o_ref[2 * j:2 * j + 2] = x_ref[2 * j:2 * j + 2] + bias


def kernel(X, table):
    x_t = jnp.transpose(X, (1, 2, 0))
    tbl = jnp.broadcast_to(table[:, :, None], (_TOT, _D, _LANE))

    bias_specs = [
        pl.BlockSpec((2, _D, _LANE), functools.partial(_bias_index_map, j))
        for j in range(_PAIRS)
    ]
    out = pl.pallas_call(
        _body,
        grid=(pl.cdiv(_F, _FB),),
        in_specs=[pl.BlockSpec((_FB, _D, _B), lambda i: (i, 0, 0))] + bias_specs,
        out_specs=pl.BlockSpec((_FB, _D, _B), lambda i: (i, 0, 0)),
        out_shape=jax.ShapeDtypeStruct((_F, _D, _B), X.dtype),
    )(x_t, *([tbl] * _PAIRS))
    return jnp.transpose(out, (2, 0, 1))


# speedup vs baseline: 6.6077x; 1.0644x over previous
import jax
import jax.numpy as jnp
from jax.experimental import pallas as pl

_TS = 26
_TOT = 126
_REP = 20
_F = _TS + (_TOT - _TS) * _REP
_D = 64
_B = 1024
_FB = 32
_PAIRS = _FB // 2
_NPAIR = _F // 2
_LANE = 128


def _body(x_ref, tbl_ref, o_ref):
    i = pl.program_id(0)
    for j in range(_PAIRS):
        p = jnp.minimum(i * _PAIRS + j, _NPAIR - 1)
        rh = jnp.where(p < _TS // 2, p, _TS // 2 + (p - _TS // 2) % 50)
        pair = tbl_ref[pl.ds(2 * rh, 2)]
        bias = jnp.concatenate([pair] * (_B // _LANE), axis=2)
        o_ref[2 * j:2 * j + 2] = x_ref[2 * j:2 * j + 2] + bias


def kernel(X, table):
    x_t = jnp.transpose(X, (1, 2, 0))
    tbl = jnp.broadcast_to(table[:, :, None], (_TOT, _D, _LANE))

    out = pl.pallas_call(
        _body,
        grid=(pl.cdiv(_F, _FB),),
        in_specs=[
            pl.BlockSpec((_FB, _D, _B), lambda i: (i, 0, 0)),
            pl.BlockSpec((_TOT, _D, _LANE), lambda i: (0, 0, 0)),
        ],
        out_specs=pl.BlockSpec((_FB, _D, _B), lambda i: (i, 0, 0)),
        out_shape=jax.ShapeDtypeStruct((_F, _D, _B), X.dtype),
    )(x_t, tbl)
    return jnp.transpose(out, (2, 0, 1))


# speedup vs baseline: 6.6601x; 1.0079x over previous
import jax
import jax.numpy as jnp
from jax.experimental import pallas as pl
from jax.experimental.pallas import tpu as pltpu

_TS = 26
_TOT = 126
_REP = 20
_F = _TS + (_TOT - _TS) * _REP
_D = 64
_B = 1024
_FB = 32
_PAIRS = _FB // 2
_NPAIR = _F // 2
_LANE = 128


def _body(x_ref, tbl_ref, o_ref, spl_ref):
    @pl.when(pl.program_id(0) == 0)
    def _init():
        t = tbl_ref[...]
        spl_ref[...] = jnp.broadcast_to(t[:, :, None], (_TOT, _D, _LANE))

    i = pl.program_id(0)
    for j in range(_PAIRS):
        p = jnp.minimum(i * _PAIRS + j, _NPAIR - 1)
        rh = jnp.where(p < _TS // 2, p, _TS // 2 + (p - _TS // 2) % 50)
        pair = spl_ref[pl.ds(2 * rh, 2)]
        bias = jnp.concatenate([pair] * (_B // _LANE), axis=2)
        o_ref[2 * j:2 * j + 2] = x_ref[2 * j:2 * j + 2] + bias


def kernel(X, table):
    x_t = jnp.transpose(X, (1, 2, 0))

    out = pl.pallas_call(
        _body,
        grid=(pl.cdiv(_F, _FB),),
        in_specs=[
            pl.BlockSpec((_FB, _D, _B), lambda i: (i, 0, 0)),
            pl.BlockSpec((_TOT, _D), lambda i: (0, 0)),
        ],
        out_specs=pl.BlockSpec((_FB, _D, _B), lambda i: (i, 0, 0)),
        out_shape=jax.ShapeDtypeStruct((_F, _D, _B), X.dtype),
        scratch_shapes=[pltpu.VMEM((_TOT, _D, _LANE), jnp.float32)],
    )(x_t, table)
    return jnp.transpose(out, (2, 0, 1))


# speedup vs baseline: 6.6785x; 1.0028x over previous
import jax
import jax.numpy as jnp
from jax.experimental import pallas as pl
from jax.experimental.pallas import tpu as pltpu

_TS = 26
_TOT = 126
_REP = 20
_F = _TS + (_TOT - _TS) * _REP
_D = 64
_B = 1024
_FB = 48
_PAIRS = _FB // 2
_NPAIR = _F // 2
_LANE = 128


def _body(x_ref, tbl_ref, o_ref, spl_ref):
    @pl.when(pl.program_id(0) == 0)
    def _init():
        t = tbl_ref[...]
        spl_ref[...] = jnp.broadcast_to(t[:, :, None], (_TOT, _D, _LANE))

    i = pl.program_id(0)
    for j in range(_PAIRS):
        p = jnp.minimum(i * _PAIRS + j, _NPAIR - 1)
        rh = jnp.where(p < _TS // 2, p, _TS // 2 + (p - _TS // 2) % 50)
        pair = spl_ref[pl.ds(2 * rh, 2)]
        bias = jnp.concatenate([pair] * (_B // _LANE), axis=2)
        o_ref[2 * j:2 * j + 2] = x_ref[2 * j:2 * j + 2] + bias


def kernel(X, table):
    x_t = jnp.transpose(X, (1, 2, 0))

    out = pl.pallas_call(
        _body,
        grid=(pl.cdiv(_F, _FB),),
        in_specs=[
            pl.BlockSpec((_FB, _D, _B), lambda i: (i, 0, 0)),
            pl.BlockSpec((_TOT, _D), lambda i: (0, 0)),
        ],
        out_specs=pl.BlockSpec((_FB, _D, _B), lambda i: (i, 0, 0)),
        out_shape=jax.ShapeDtypeStruct((_F, _D, _B), X.dtype),
        scratch_shapes=[pltpu.VMEM((_TOT, _D, _LANE), jnp.float32)],
    )(x_t, table)
    return jnp.transpose(out, (2, 0, 1))


# speedup vs baseline: 6.6822x; 1.0005x over previous
import jax
import jax.numpy as jnp
from jax.experimental import pallas as pl
from jax.experimental.pallas import tpu as pltpu

_TS = 26
_TOT = 126
_REP = 20
_F = _TS + (_TOT - _TS) * _REP
_D = 64
_B = 1024
_HB = _B // 2
_FB = 48
_PAIRS = _FB // 2
_NPAIR = _F // 2
_LANE = 128


def _body(xlo_ref, xhi_ref, tbl_ref, o_ref, spl_ref):
    @pl.when(pl.program_id(0) == 0)
    def _init():
        t = tbl_ref[...]
        spl_ref[...] = jnp.broadcast_to(t[:, :, None], (_TOT, _D, _LANE))

    i = pl.program_id(0)
    for j in range(_PAIRS):
        p = jnp.minimum(i * _PAIRS + j, _NPAIR - 1)
        rh = jnp.where(p < _TS // 2, p, _TS // 2 + (p - _TS // 2) % 50)
        pair = spl_ref[pl.ds(2 * rh, 2)]
        bias = jnp.concatenate([pair] * (_HB // _LANE), axis=2)
        o_ref[2 * j:2 * j + 2, :, 0:_HB] = xlo_ref[2 * j:2 * j + 2] + bias
        o_ref[2 * j:2 * j + 2, :, _HB:_B] = xhi_ref[2 * j:2 * j + 2] + bias


def kernel(X, table):
    x_t = jnp.transpose(X, (1, 2, 0))

    out = pl.pallas_call(
        _body,
        grid=(pl.cdiv(_F, _FB),),
        in_specs=[
            pl.BlockSpec((_FB, _D, _HB), lambda i: (i, 0, 0)),
            pl.BlockSpec((_FB, _D, _HB), lambda i: (i, 0, 1)),
            pl.BlockSpec((_TOT, _D), lambda i: (0, 0)),
        ],
        out_specs=pl.BlockSpec((_FB, _D, _B), lambda i: (i, 0, 0)),
        out_shape=jax.ShapeDtypeStruct((_F, _D, _B), X.dtype),
        scratch_shapes=[pltpu.VMEM((_TOT, _D, _LANE), jnp.float32)],
    )(x_t, x_t, table)
    return jnp.transpose(out, (2, 0, 1))
